# Initial kernel scaffold; baseline (speedup 1.0000x reference)
import jax
import jax.numpy as jnp
from jax.experimental import pallas as pl


def kernel(seqs, exps, W_seq, W_exp):
    raise NotImplementedError("write your pallas kernel here")


# speedup vs baseline: 4.2762x; 4.2762x over previous
import functools

import jax
import jax.numpy as jnp
from jax import lax
from jax.experimental import pallas as pl
from jax.experimental.pallas import tpu as pltpu
from jax.experimental.pallas import tpu_sc as plsc

EMB = 64
IDX_ROW = 128
ROWS_PER_CHUNK = 4
CHUNK = IDX_ROW * ROWS_PER_CHUNK


@functools.cache
def _build(n_idx: int):
    info = plsc.get_sparse_core_info()
    nw = info.num_cores * info.num_subcores
    n_chunks = n_idx // CHUNK
    assert n_chunks * CHUNK == n_idx

    mesh = plsc.VectorSubcoreMesh(core_axis_name="c", subcore_axis_name="s")
    out_t = jax.ShapeDtypeStruct((n_idx, EMB), jnp.float32)

    @functools.partial(
        pl.kernel,
        mesh=mesh,
        out_type=[out_t, out_t],
        scratch_types=[
            pltpu.VMEM((ROWS_PER_CHUNK, IDX_ROW), jnp.int32),
            pltpu.VMEM((CHUNK, EMB), jnp.float32),
            pltpu.SemaphoreType.DMA,
        ],
        compiler_params=pltpu.CompilerParams(use_tc_tiling_on_sc=False),
    )
    def k(w_seq, w_exp, seq_idx, exp_idx, o_seq, o_exp, idx_v, rows_v, sem):
        wid = lax.axis_index("s") * info.num_cores + lax.axis_index("c")

        def do_table(idx_hbm, w_hbm, out_hbm):
            def body(j, carry):
                c = wid + j * nw
                pltpu.sync_copy(idx_hbm.at[pl.ds(c * ROWS_PER_CHUNK, ROWS_PER_CHUNK)], idx_v)
                cps = [
                    pltpu.async_copy(
                        w_hbm.at[idx_v.at[i]],
                        rows_v.at[pl.ds(i * IDX_ROW, IDX_ROW)],
                        sem,
                    )
                    for i in range(ROWS_PER_CHUNK)
                ]
                for cp in cps:
                    cp.wait()
                pltpu.sync_copy(rows_v, out_hbm.at[pl.ds(c * CHUNK, CHUNK)])
                return carry

            n_mine = (n_chunks - wid + nw - 1) // nw
            lax.fori_loop(0, n_mine, body, 0)

        do_table(seq_idx, w_seq, o_seq)
        do_table(exp_idx, w_exp, o_exp)

    return k


def kernel(seqs, exps, W_seq, W_exp):
    b, s = seqs.shape
    n_idx = b * s
    seq_idx = seqs.reshape(n_idx // IDX_ROW, IDX_ROW).astype(jnp.int32)
    exp_idx = exps.reshape(n_idx // IDX_ROW, IDX_ROW).astype(jnp.int32)
    o_seq, o_exp = _build(n_idx)(W_seq, W_exp, seq_idx, exp_idx)
    return (o_seq.reshape(b, s, EMB), o_exp.reshape(b, s, EMB))
